# baseline (speedup 1.0000x reference)
import functools

import jax
import jax.numpy as jnp
from jax import lax
from jax.experimental import pallas as pl
from jax.experimental.pallas import tpu as pltpu
from jax.experimental.pallas import tpu_sc as plsc

B, T, D_IN = 64, 1024, 192
K, D_EMB = 512, 32
M = B * T

BM1 = 2048
BM3 = 4096

_NC, _NS = 2, 16
_NW = _NC * _NS
_BPW = M // _NW
_CHUNK = 128


def _enc_argmin_body(x_ref, we_ref, be_ref, cb_ref, ze_ref, idx_ref):
    xb = x_ref[...]
    z = jnp.dot(xb, we_ref[...]) + be_ref[...]
    ze_ref[...] = z
    cb = cb_ref[...]
    mm = lax.dot_general(z, cb, (((1,), (1,)), ((), ())))
    zs = jnp.sum(z * z, axis=1, keepdims=True)
    csq = jnp.sum(cb * cb, axis=1)
    d = zs - 2.0 * mm + csq[None, :]
    mval = jnp.min(d, axis=1, keepdims=True)
    ii = lax.broadcasted_iota(jnp.int32, d.shape, 1)
    sel = jnp.where(d == mval, ii, d.shape[1])
    idx = jnp.min(sel, axis=1, keepdims=True)
    idx_ref[...] = idx.reshape(BM1 // 128, 128)


def _dec_body(zq4_ref, wd_ref, bd_ref, xr_ref, zq_ref):
    zq = zq4_ref[...][:, :D_EMB]
    zq_ref[...] = zq
    xr_ref[...] = jnp.dot(zq, wd_ref[...]) + bd_ref[...]


def _sc_gather(cb_pad, idx_flat):
    mesh = plsc.VectorSubcoreMesh(core_axis_name="c", subcore_axis_name="s")
    n_chunks = _BPW // _CHUNK

    @functools.partial(
        pl.kernel,
        mesh=mesh,
        out_type=jax.ShapeDtypeStruct((M, 128), jnp.float32),
        scratch_types=[
            pltpu.VMEM((_BPW // _CHUNK, _CHUNK), jnp.int32),
            pltpu.VMEM((_CHUNK, 128), jnp.float32),
            pltpu.VMEM((_CHUNK, 128), jnp.float32),
            pltpu.SemaphoreType.DMA,
            pltpu.SemaphoreType.DMA,
        ],
    )
    def gather_kernel(cb_hbm, idx_hbm, out_hbm, idx_v, stage_a, stage_b, sem_a, sem_b):
        wid = lax.axis_index("s") * _NC + lax.axis_index("c")
        base = wid * _BPW
        pltpu.sync_copy(idx_hbm.at[pl.ds(wid * (_BPW // _CHUNK), _BPW // _CHUNK)], idx_v)
        stages = (stage_a, stage_b)
        sems = (sem_a, sem_b)

        def start(j):
            return pltpu.async_copy(
                cb_hbm.at[idx_v.at[j]],
                stages[j % 2],
                sems[j % 2],
            )
        pending = start(0)
        for j in range(n_chunks):
            nxt = start(j + 1) if j + 1 < n_chunks else None
            pending.wait()
            pltpu.sync_copy(stages[j % 2], out_hbm.at[pl.ds(base + j * _CHUNK, _CHUNK)])
            pending = nxt

    return gather_kernel(cb_pad, idx_flat)


def kernel(x, W_enc, b_enc, codebook, W_dec, b_dec):
    x2 = x.reshape(M, D_IN)

    z_e, idx = pl.pallas_call(
        _enc_argmin_body,
        grid=(M // BM1,),
        in_specs=[
            pl.BlockSpec((BM1, D_IN), lambda i: (i, 0)),
            pl.BlockSpec((D_IN, D_EMB), lambda i: (0, 0)),
            pl.BlockSpec((1, D_EMB), lambda i: (0, 0)),
            pl.BlockSpec((K, D_EMB), lambda i: (0, 0)),
        ],
        out_specs=[
            pl.BlockSpec((BM1, D_EMB), lambda i: (i, 0)),
            pl.BlockSpec((BM1 // 128, 128), lambda i: (i, 0)),
        ],
        out_shape=[
            jax.ShapeDtypeStruct((M, D_EMB), jnp.float32),
            jax.ShapeDtypeStruct((M // 128, 128), jnp.int32),
        ],
    )(x2, W_enc, b_enc.reshape(1, D_EMB), codebook)

    cb_pad = jnp.pad(codebook, ((0, 0), (0, 128 - D_EMB)))
    zq4 = _sc_gather(cb_pad, idx)

    x_rec, z_q = pl.pallas_call(
        _dec_body,
        grid=(M // BM3,),
        in_specs=[
            pl.BlockSpec((BM3, 128), lambda i: (i, 0)),
            pl.BlockSpec((D_EMB, D_IN), lambda i: (0, 0)),
            pl.BlockSpec((1, D_IN), lambda i: (0, 0)),
        ],
        out_specs=[
            pl.BlockSpec((BM3, D_IN), lambda i: (i, 0)),
            pl.BlockSpec((BM3, D_EMB), lambda i: (i, 0)),
        ],
        out_shape=[
            jax.ShapeDtypeStruct((M, D_IN), jnp.float32),
            jax.ShapeDtypeStruct((M, D_EMB), jnp.float32),
        ],
    )(zq4, W_dec, b_dec.reshape(1, D_IN))

    return (
        x_rec.reshape(B, T, D_IN),
        z_e.reshape(B, T, D_EMB),
        z_q.reshape(B, T, D_EMB),
    )


# speedup vs baseline: 1.5206x; 1.5206x over previous
import functools

import jax
import jax.numpy as jnp
from jax import lax
from jax.experimental import pallas as pl
from jax.experimental.pallas import tpu as pltpu
from jax.experimental.pallas import tpu_sc as plsc

B, T, D_IN = 64, 1024, 192
K, D_EMB = 512, 32
M = B * T

BM1 = 2048
BM3 = 2048

_NC, _NS = 2, 16
_NW = _NC * _NS
_BPW = M // _NW
_CHUNK = 128


def _enc_argmin_body(x_ref, we_ref, be_ref, cb_ref, ze_ref, idxp_ref, idxc_ref):
    xb = x_ref[...]
    z = jnp.dot(xb, we_ref[...]) + be_ref[...]
    ze_ref[...] = z
    cb = cb_ref[...]
    mm = lax.dot_general(z, cb, (((1,), (1,)), ((), ())))
    zs = jnp.sum(z * z, axis=1, keepdims=True)
    csq = jnp.sum(cb * cb, axis=1)
    d = zs - 2.0 * mm + csq[None, :]
    mval = jnp.min(d, axis=1, keepdims=True)
    ii = lax.broadcasted_iota(jnp.int32, d.shape, 1)
    sel = jnp.where(d == mval, ii, d.shape[1])
    idx = jnp.min(sel, axis=1, keepdims=True)
    idxc_ref[...] = idx
    idxp_ref[...] = idx.reshape(BM1 // 128, 128)


def _dec_body(idx_ref, cb_ref, wd_ref, bd_ref, xr_ref):
    idx = idx_ref[...]
    ii = lax.broadcasted_iota(jnp.int32, (BM3, K), 1)
    oh = (ii == idx).astype(jnp.float32)
    q = jnp.dot(oh, cb_ref[...])
    xr_ref[...] = jnp.dot(q, wd_ref[...]) + bd_ref[...]


def _sc_gather(codebook, idx_packed):
    mesh = plsc.VectorSubcoreMesh(core_axis_name="c", subcore_axis_name="s")
    n_chunks = _BPW // _CHUNK
    rows_per_w = _BPW // _CHUNK

    @functools.partial(
        pl.kernel,
        mesh=mesh,
        out_type=jax.ShapeDtypeStruct((M, D_EMB), jnp.float32),
        compiler_params=pltpu.CompilerParams(use_tc_tiling_on_sc=False),
        scratch_types=[
            pltpu.VMEM((rows_per_w, _CHUNK), jnp.int32),
            pltpu.VMEM((_BPW, D_EMB), jnp.float32),
            pltpu.SemaphoreType.DMA,
        ],
    )
    def gather_kernel(cb_hbm, idx_hbm, out_hbm, idx_v, rows_v, sem):
        wid = lax.axis_index("s") * _NC + lax.axis_index("c")
        base = wid * _BPW
        pltpu.sync_copy(idx_hbm.at[pl.ds(wid * rows_per_w, rows_per_w)], idx_v)
        copies = [
            pltpu.async_copy(
                cb_hbm.at[idx_v.at[j]],
                rows_v.at[pl.ds(j * _CHUNK, _CHUNK)],
                sem,
            )
            for j in range(n_chunks)
        ]
        for c in copies:
            c.wait()
        pltpu.sync_copy(rows_v, out_hbm.at[pl.ds(base, _BPW)])

    return gather_kernel(codebook, idx_packed)


def kernel(x, W_enc, b_enc, codebook, W_dec, b_dec):
    x2 = x.reshape(M, D_IN)

    z_e, idx_packed, idx_col = pl.pallas_call(
        _enc_argmin_body,
        grid=(M // BM1,),
        in_specs=[
            pl.BlockSpec((BM1, D_IN), lambda i: (i, 0)),
            pl.BlockSpec((D_IN, D_EMB), lambda i: (0, 0)),
            pl.BlockSpec((1, D_EMB), lambda i: (0, 0)),
            pl.BlockSpec((K, D_EMB), lambda i: (0, 0)),
        ],
        out_specs=[
            pl.BlockSpec((BM1, D_EMB), lambda i: (i, 0)),
            pl.BlockSpec((BM1 // 128, 128), lambda i: (i, 0)),
            pl.BlockSpec((BM1, 1), lambda i: (i, 0)),
        ],
        out_shape=[
            jax.ShapeDtypeStruct((M, D_EMB), jnp.float32),
            jax.ShapeDtypeStruct((M // 128, 128), jnp.int32),
            jax.ShapeDtypeStruct((M, 1), jnp.int32),
        ],
    )(x2, W_enc, b_enc.reshape(1, D_EMB), codebook)

    z_q = _sc_gather(codebook, idx_packed)

    x_rec = pl.pallas_call(
        _dec_body,
        grid=(M // BM3,),
        in_specs=[
            pl.BlockSpec((BM3, 1), lambda i: (i, 0)),
            pl.BlockSpec((K, D_EMB), lambda i: (0, 0)),
            pl.BlockSpec((D_EMB, D_IN), lambda i: (0, 0)),
            pl.BlockSpec((1, D_IN), lambda i: (0, 0)),
        ],
        out_specs=pl.BlockSpec((BM3, D_IN), lambda i: (i, 0)),
        out_shape=jax.ShapeDtypeStruct((M, D_IN), jnp.float32),
    )(idx_col, codebook, W_dec, b_dec.reshape(1, D_IN))

    return (
        x_rec.reshape(B, T, D_IN),
        z_e.reshape(B, T, D_EMB),
        z_q.reshape(B, T, D_EMB),
    )


# speedup vs baseline: 1.5236x; 1.0020x over previous
import functools

import jax
import jax.numpy as jnp
from jax import lax
from jax.experimental import pallas as pl
from jax.experimental.pallas import tpu as pltpu
from jax.experimental.pallas import tpu_sc as plsc

B, T, D_IN = 64, 1024, 192
K, D_EMB = 512, 32
M = B * T

BM1 = 2048
BM3 = 2048

_NC, _NS = 2, 16
_NW = _NC * _NS
_BPW = M // _NW
_CHUNK = 128


def _enc_argmin_body(x_ref, we_ref, be_ref, cb_ref, ze_ref, idxp_ref, idxc_ref):
    xb = x_ref[...]
    z = jnp.dot(xb, we_ref[...]) + be_ref[...]
    ze_ref[...] = z
    cb = cb_ref[...]
    mm = lax.dot_general(z, cb, (((1,), (1,)), ((), ())))
    zs = jnp.sum(z * z, axis=1, keepdims=True)
    csq = jnp.sum(cb * cb, axis=1)
    d = zs - 2.0 * mm + csq[None, :]
    mval = jnp.min(d, axis=1, keepdims=True)
    ii = lax.broadcasted_iota(jnp.int32, d.shape, 1)
    sel = jnp.where(d == mval, ii, d.shape[1])
    idx = jnp.min(sel, axis=1, keepdims=True)
    idxc_ref[...] = idx
    idxp_ref[...] = idx.reshape(BM1 // 128, 128)


def _dec_body(idx_ref, cb_ref, wd_ref, bd_ref, xr_ref):
    idx = idx_ref[...]
    ii = lax.broadcasted_iota(jnp.int32, (BM3, K), 1)
    oh = (ii == idx).astype(jnp.float32)
    q = jnp.dot(oh, cb_ref[...])
    xr_ref[...] = jnp.dot(q, wd_ref[...]) + bd_ref[...]


_NCHUNK = _BPW // _CHUNK


def _sc_gather(cb, idx_packed):
    mesh = plsc.VectorSubcoreMesh(core_axis_name="c", subcore_axis_name="s")

    @functools.partial(
        pl.kernel,
        mesh=mesh,
        out_type=jax.ShapeDtypeStruct((M, D_EMB), jnp.float32),
        compiler_params=pltpu.CompilerParams(use_tc_tiling_on_sc=False),
        scratch_types=[
            pltpu.VMEM((_NCHUNK, _CHUNK), jnp.int32),
            pltpu.VMEM((_BPW, D_EMB), jnp.float32),
            pltpu.SemaphoreType.DMA,
        ],
    )
    def gather_kernel(cb_hbm, idx_hbm, out_hbm, idx_v, rows_v, sem):
        wid = lax.axis_index("s") * _NC + lax.axis_index("c")
        base = wid * _BPW
        pltpu.sync_copy(idx_hbm.at[pl.ds(wid * _NCHUNK, _NCHUNK)], idx_v)
        copies = [
            pltpu.async_copy(
                cb_hbm.at[idx_v.at[c]],
                rows_v.at[pl.ds(c * _CHUNK, _CHUNK)],
                sem,
            )
            for c in range(_NCHUNK)
        ]
        for cp in copies:
            cp.wait()
        pltpu.sync_copy(rows_v, out_hbm.at[pl.ds(base, _BPW)])

    return gather_kernel(cb, idx_packed)


def kernel(x, W_enc, b_enc, codebook, W_dec, b_dec):
    x2 = x.reshape(M, D_IN)

    z_e, idx_packed, idx_col = pl.pallas_call(
        _enc_argmin_body,
        grid=(M // BM1,),
        in_specs=[
            pl.BlockSpec((BM1, D_IN), lambda i: (i, 0)),
            pl.BlockSpec((D_IN, D_EMB), lambda i: (0, 0)),
            pl.BlockSpec((1, D_EMB), lambda i: (0, 0)),
            pl.BlockSpec((K, D_EMB), lambda i: (0, 0)),
        ],
        out_specs=[
            pl.BlockSpec((BM1, D_EMB), lambda i: (i, 0)),
            pl.BlockSpec((BM1 // 128, 128), lambda i: (i, 0)),
            pl.BlockSpec((BM1, 1), lambda i: (i, 0)),
        ],
        out_shape=[
            jax.ShapeDtypeStruct((M, D_EMB), jnp.float32),
            jax.ShapeDtypeStruct((M // 128, 128), jnp.int32),
            jax.ShapeDtypeStruct((M, 1), jnp.int32),
        ],
    )(x2, W_enc, b_enc.reshape(1, D_EMB), codebook)

    z_q = _sc_gather(codebook, idx_packed)

    x_rec = pl.pallas_call(
        _dec_body,
        grid=(M // BM3,),
        in_specs=[
            pl.BlockSpec((BM3, 1), lambda i: (i, 0)),
            pl.BlockSpec((K, D_EMB), lambda i: (0, 0)),
            pl.BlockSpec((D_EMB, D_IN), lambda i: (0, 0)),
            pl.BlockSpec((1, D_IN), lambda i: (0, 0)),
        ],
        out_specs=pl.BlockSpec((BM3, D_IN), lambda i: (i, 0)),
        out_shape=jax.ShapeDtypeStruct((M, D_IN), jnp.float32),
    )(idx_col, codebook, W_dec, b_dec.reshape(1, D_IN))

    return (
        x_rec.reshape(B, T, D_IN),
        z_e.reshape(B, T, D_EMB),
        z_q.reshape(B, T, D_EMB),
    )
